# baseline (speedup 1.0000x reference)
import jax
import jax.numpy as jnp
from jax import lax
from jax.experimental import pallas as pl
from jax.experimental.pallas import tpu as pltpu
from jax.experimental.pallas import tpu_sc as plsc

N = 100000
D = 128
G = 512
NW = 32
RW = 3200
C = 160
NCH = RW // C
L = 16


def _sc_body(h_hbm, seg_hbm, w_hbm, b_hbm, num_hbm, den_hbm,
             hb0, hb1, sb0, sb1, wv, bv, num_acc, den_acc,
             sh0, sh1, ss0, ss1):
    cid = lax.axis_index("c")
    sid = lax.axis_index("s")
    wid = cid * 16 + sid
    r0 = wid * RW

    pltpu.sync_copy(w_hbm, wv)
    pltpu.sync_copy(b_hbm, bv)

    def _zero(r, carry):
        for k in range(D // L):
            num_acc[r, pl.ds(L * k, L)] = jnp.zeros((L,), jnp.float32)
        den_acc[pl.ds(r * L, L)] = jnp.zeros((L,), jnp.float32)
        return carry
    lax.fori_loop(0, G, _zero, None)

    wk = [wv[pl.ds(L * k, L)] for k in range(D // L)]
    bsc = bv[...][0]

    def _chunk_base(j):
        s_j = r0 + j * C
        d_j = jnp.minimum(s_j, N - C)
        return s_j, d_j

    def _dma_start(j, hb, sb, sh, ss):
        _, d_j = _chunk_base(j)
        pltpu.async_copy(h_hbm.at[pl.ds(d_j, C), :], hb, sh)
        pltpu.async_copy(seg_hbm.at[pl.ds(d_j, C)], sb, ss)

    def _dma_wait(j, hb, sb, sh, ss):
        _, d_j = _chunk_base(j)
        pltpu.make_async_copy(h_hbm.at[pl.ds(d_j, C), :], hb, sh).wait()
        pltpu.make_async_copy(seg_hbm.at[pl.ds(d_j, C)], sb, ss).wait()

    def _process(j, hb, sb):
        s_j, d_j = _chunk_base(j)
        lo = s_j - d_j
        hi = jnp.minimum(s_j + C, N) - d_j

        def _group(g, carry):
            sv = sb[pl.ds(g * L, L)]
            row_vec = g * L + lax.iota(jnp.int32, L)

            parts = []
            for q in range(D // L):
                pa = None
                for l in range(L):
                    dcol = q * L + l
                    hcol = plsc.load_gather(
                        hb, [row_vec, jnp.full((L,), dcol, jnp.int32)])
                    t = hcol * wk[q][l]
                    pa = t if pa is None else pa + t
                parts.append(pa)
            while len(parts) > 1:
                parts = [parts[i] + parts[i + 1]
                         for i in range(0, len(parts) - 1, 2)] + (
                    [parts[-1]] if len(parts) % 2 else [])
            a16 = parts[0] + bsc
            a16 = jnp.where((row_vec >= lo) & (row_vec < hi), a16, -1e30)
            att16 = jnp.exp(jnp.where(a16 >= 0, a16, 0.01 * a16))

            def _loads(k):
                r = g * L + k
                return [hb[r, pl.ds(L * q, L)] for q in range(D // L)]

            hk = _loads(0)
            for k in range(L):
                s = sv[k]
                att = att16[k]
                av = jnp.full((L,), att, jnp.float32)
                pk = [av * hk[q] for q in range(D // L)]
                hk = _loads(k + 1) if k + 1 < L else None
                for q in range(D // L):
                    plsc.addupdate(num_acc.at[s, pl.ds(L * q, L)], pk[q])
                plsc.addupdate(den_acc.at[pl.ds(s * L, L)], av)
            return carry
        lax.fori_loop(0, C // L, _group, None)

    _dma_start(0, hb0, sb0, sh0, ss0)

    def _outer(i, carry):
        j0 = 2 * i
        j1 = 2 * i + 1
        _dma_wait(j0, hb0, sb0, sh0, ss0)
        _dma_start(j1, hb1, sb1, sh1, ss1)
        _process(j0, hb0, sb0)
        _dma_wait(j1, hb1, sb1, sh1, ss1)
        _dma_start(jnp.minimum(j1 + 1, NCH - 1), hb0, sb0, sh0, ss0)
        _process(j1, hb1, sb1)
        return carry
    lax.fori_loop(0, NCH // 2, _outer, None)

    _dma_wait(NCH - 1, hb0, sb0, sh0, ss0)

    pltpu.sync_copy(num_acc, num_hbm.at[wid])
    pltpu.sync_copy(den_acc, den_hbm.at[wid])


GB = 64


def _tc_merge(p_ref, d_ref, o_ref):
    num = jnp.sum(p_ref[...], axis=0)
    dsum = jnp.sum(d_ref[...], axis=0)
    den_rep = jnp.reshape(
        jnp.broadcast_to(dsum[:, None, :], (GB // 8, 8, D)), (GB, D))
    rows = lax.broadcasted_iota(jnp.int32, (GB, D), 0)
    lanes = lax.broadcasted_iota(jnp.int32, (GB, D), 1)
    mask = lanes == (rows % 8) * L
    den = jnp.sum(jnp.where(mask, den_rep, 0.0), axis=1, keepdims=True)
    o_ref[...] = num / jnp.maximum(den, 1e-9)


def kernel(h, segment_ids, W, b):
    seg = segment_ids.astype(jnp.int32)
    wf = W.reshape(D).astype(jnp.float32)
    bf = jnp.broadcast_to(b.astype(jnp.float32).reshape(1), (L,))
    mesh = plsc.VectorSubcoreMesh(core_axis_name="c", subcore_axis_name="s")
    sc = pl.kernel(
        _sc_body,
        mesh=mesh,
        compiler_params=pltpu.CompilerParams(needs_layout_passes=False),
        out_type=(
            jax.ShapeDtypeStruct((NW, G, D), jnp.float32),
            jax.ShapeDtypeStruct((NW, G * L), jnp.float32),
        ),
        scratch_types=[
            pltpu.VMEM((C, D), jnp.float32),
            pltpu.VMEM((C, D), jnp.float32),
            pltpu.VMEM((C,), jnp.int32),
            pltpu.VMEM((C,), jnp.int32),
            pltpu.VMEM((D,), jnp.float32),
            pltpu.VMEM((L,), jnp.float32),
            pltpu.VMEM((G, D), jnp.float32),
            pltpu.VMEM((G * L,), jnp.float32),
            pltpu.SemaphoreType.DMA,
            pltpu.SemaphoreType.DMA,
            pltpu.SemaphoreType.DMA,
            pltpu.SemaphoreType.DMA,
        ],
    )
    num_p, den_p = sc(h, seg, wf, bf)
    den3 = den_p.reshape(NW, G // 8, D)
    out = pl.pallas_call(
        _tc_merge,
        grid=(G // GB,),
        in_specs=[
            pl.BlockSpec((NW, GB, D), lambda i: (0, i, 0)),
            pl.BlockSpec((NW, GB // 8, D), lambda i: (0, i, 0)),
        ],
        out_specs=pl.BlockSpec((GB, D), lambda i: (i, 0)),
        out_shape=jax.ShapeDtypeStruct((G, D), jnp.float32),
    )(num_p, den3)
    return out


# speedup vs baseline: 2.6067x; 2.6067x over previous
import jax
import jax.numpy as jnp
from jax import lax
from jax.experimental import pallas as pl
from jax.experimental.pallas import tpu as pltpu
from jax.experimental.pallas import tpu_sc as plsc

N = 100000
D = 128
G = 512
NW = 32
RW = 3200
C = 160
NCH = RW // C
L = 16
BN = 4096
NATT = ((N + BN - 1) // BN) * BN


def _tc_att(h_ref, w_ref, b_ref, o_ref):
    z8 = lax.dot_general(w_ref[...], h_ref[...],
                         (((1,), (1,)), ((), ())),
                         preferred_element_type=jnp.float32)
    z = z8[:1, :] + b_ref[0, 0]
    o_ref[...] = jnp.exp(jnp.where(z >= 0, z, 0.01 * z)).reshape(BN)


def _sc_body(h_hbm, seg_hbm, att_hbm, num_hbm, den_hbm,
             hb0, hb1, sb0, sb1, ab0, ab1, num_acc, den_acc,
             sh0, sh1, ss0, ss1, sa0, sa1):
    cid = lax.axis_index("c")
    sid = lax.axis_index("s")
    wid = cid * 16 + sid
    r0 = wid * RW

    def _zero(r, carry):
        for k in range(D // L):
            num_acc[r, pl.ds(L * k, L)] = jnp.zeros((L,), jnp.float32)
        den_acc[pl.ds(r * L, L)] = jnp.zeros((L,), jnp.float32)
        return carry
    lax.fori_loop(0, G, _zero, None)

    def _chunk_base(j):
        s_j = r0 + j * C
        d_j = jnp.minimum(s_j, N - C)
        return s_j, d_j

    def _dma_start(j, hb, sb, ab, sh, ss, sa):
        _, d_j = _chunk_base(j)
        pltpu.async_copy(h_hbm.at[pl.ds(d_j, C), :], hb, sh)
        pltpu.async_copy(seg_hbm.at[pl.ds(d_j, C)], sb, ss)
        pltpu.async_copy(att_hbm.at[pl.ds(d_j, C)], ab, sa)

    def _dma_wait(j, hb, sb, ab, sh, ss, sa):
        _, d_j = _chunk_base(j)
        pltpu.make_async_copy(h_hbm.at[pl.ds(d_j, C), :], hb, sh).wait()
        pltpu.make_async_copy(seg_hbm.at[pl.ds(d_j, C)], sb, ss).wait()
        pltpu.make_async_copy(att_hbm.at[pl.ds(d_j, C)], ab, sa).wait()

    def _process(j, hb, sb, ab):
        s_j, d_j = _chunk_base(j)
        lo = s_j - d_j
        hi = jnp.minimum(s_j + C, N) - d_j

        def _group(g, carry):
            sv = sb[pl.ds(g * L, L)]
            at16 = ab[pl.ds(g * L, L)]

            def _loads(k):
                r = g * L + k
                return [hb[r, pl.ds(L * q, L)] for q in range(D // L)]

            hk = _loads(0)
            for k in range(L):
                r = g * L + k
                s = sv[k]
                att = jnp.where((r >= lo) & (r < hi), at16[k], 0.0)
                av = jnp.full((L,), att, jnp.float32)
                pk = [av * hk[q] for q in range(D // L)]
                hk = _loads(k + 1) if k + 1 < L else None
                for q in range(D // L):
                    plsc.addupdate(num_acc.at[s, pl.ds(L * q, L)], pk[q])
                plsc.addupdate(den_acc.at[pl.ds(s * L, L)], av)
            return carry
        lax.fori_loop(0, C // L, _group, None)

    _dma_start(0, hb0, sb0, ab0, sh0, ss0, sa0)

    def _outer(i, carry):
        j0 = 2 * i
        j1 = 2 * i + 1
        _dma_wait(j0, hb0, sb0, ab0, sh0, ss0, sa0)
        _dma_start(j1, hb1, sb1, ab1, sh1, ss1, sa1)
        _process(j0, hb0, sb0, ab0)
        _dma_wait(j1, hb1, sb1, ab1, sh1, ss1, sa1)
        _dma_start(jnp.minimum(j1 + 1, NCH - 1), hb0, sb0, ab0, sh0, ss0, sa0)
        _process(j1, hb1, sb1, ab1)
        return carry
    lax.fori_loop(0, NCH // 2, _outer, None)

    _dma_wait(NCH - 1, hb0, sb0, ab0, sh0, ss0, sa0)

    pltpu.sync_copy(num_acc, num_hbm.at[wid])
    pltpu.sync_copy(den_acc, den_hbm.at[wid])


GB = 64


def _tc_merge(p_ref, d_ref, o_ref):
    num = jnp.sum(p_ref[...], axis=0)
    dsum = jnp.sum(d_ref[...], axis=0)
    den_rep = jnp.reshape(
        jnp.broadcast_to(dsum[:, None, :], (GB // 8, 8, D)), (GB, D))
    rows = lax.broadcasted_iota(jnp.int32, (GB, D), 0)
    lanes = lax.broadcasted_iota(jnp.int32, (GB, D), 1)
    mask = lanes == (rows % 8) * L
    den = jnp.sum(jnp.where(mask, den_rep, 0.0), axis=1, keepdims=True)
    o_ref[...] = num / jnp.maximum(den, 1e-9)


def kernel(h, segment_ids, W, b):
    seg = segment_ids.astype(jnp.int32)
    wf = jnp.broadcast_to(W.astype(jnp.float32), (8, D))
    bf = b.astype(jnp.float32).reshape(1, 1)
    att = pl.pallas_call(
        _tc_att,
        grid=(NATT // BN,),
        in_specs=[
            pl.BlockSpec((BN, D), lambda i: (i, 0)),
            pl.BlockSpec((8, D), lambda i: (0, 0)),
            pl.BlockSpec((1, 1), lambda i: (0, 0)),
        ],
        out_specs=pl.BlockSpec((BN,), lambda i: (i,)),
        out_shape=jax.ShapeDtypeStruct((NATT,), jnp.float32),
    )(h, wf, bf)

    mesh = plsc.VectorSubcoreMesh(core_axis_name="c", subcore_axis_name="s")
    sc = pl.kernel(
        _sc_body,
        mesh=mesh,
        compiler_params=pltpu.CompilerParams(needs_layout_passes=False),
        out_type=(
            jax.ShapeDtypeStruct((NW, G, D), jnp.float32),
            jax.ShapeDtypeStruct((NW, G * L), jnp.float32),
        ),
        scratch_types=[
            pltpu.VMEM((C, D), jnp.float32),
            pltpu.VMEM((C, D), jnp.float32),
            pltpu.VMEM((C,), jnp.int32),
            pltpu.VMEM((C,), jnp.int32),
            pltpu.VMEM((C,), jnp.float32),
            pltpu.VMEM((C,), jnp.float32),
            pltpu.VMEM((G, D), jnp.float32),
            pltpu.VMEM((G * L,), jnp.float32),
            pltpu.SemaphoreType.DMA,
            pltpu.SemaphoreType.DMA,
            pltpu.SemaphoreType.DMA,
            pltpu.SemaphoreType.DMA,
            pltpu.SemaphoreType.DMA,
            pltpu.SemaphoreType.DMA,
        ],
    )
    num_p, den_p = sc(h, seg, att)
    den3 = den_p.reshape(NW, G // 8, D)
    out = pl.pallas_call(
        _tc_merge,
        grid=(G // GB,),
        in_specs=[
            pl.BlockSpec((NW, GB, D), lambda i: (0, i, 0)),
            pl.BlockSpec((NW, GB // 8, D), lambda i: (0, i, 0)),
        ],
        out_specs=pl.BlockSpec((GB, D), lambda i: (i, 0)),
        out_shape=jax.ShapeDtypeStruct((G, D), jnp.float32),
    )(num_p, den3)
    return out


# speedup vs baseline: 2.6558x; 1.0188x over previous
import jax
import jax.numpy as jnp
from jax import lax
from jax.experimental import pallas as pl
from jax.experimental.pallas import tpu as pltpu
from jax.experimental.pallas import tpu_sc as plsc

N = 100000
D = 128
G = 512
NW = 32
RW = 3200
C = 160
NCH = RW // C
L = 16
BN = 4096
NATT = ((N + BN - 1) // BN) * BN


def _tc_att(h_ref, w_ref, b_ref, o_ref):
    z8 = lax.dot_general(w_ref[...], h_ref[...],
                         (((1,), (1,)), ((), ())),
                         preferred_element_type=jnp.float32)
    z = z8[:1, :] + b_ref[0, 0]
    o_ref[...] = jnp.exp(jnp.where(z >= 0, z, 0.01 * z)).reshape(BN)


def _sc_body(h_hbm, seg_hbm, att_hbm, num_hbm, den_hbm,
             hb0, hb1, sb0, sb1, ab0, ab1, num_acc, den_acc,
             sh0, sh1, ss0, ss1, sa0, sa1):
    cid = lax.axis_index("c")
    sid = lax.axis_index("s")
    wid = cid * 16 + sid
    r0 = wid * RW

    def _chunk_base(j):
        s_j = r0 + j * C
        d_j = jnp.minimum(s_j, N - C)
        return s_j, d_j

    def _dma_start(j, hb, sb, ab, sh, ss, sa):
        _, d_j = _chunk_base(j)
        pltpu.async_copy(h_hbm.at[pl.ds(d_j, C), :], hb, sh)
        pltpu.async_copy(seg_hbm.at[pl.ds(d_j, C)], sb, ss)
        pltpu.async_copy(att_hbm.at[pl.ds(d_j, C)], ab, sa)

    def _dma_wait(j, hb, sb, ab, sh, ss, sa):
        _, d_j = _chunk_base(j)
        pltpu.make_async_copy(h_hbm.at[pl.ds(d_j, C), :], hb, sh).wait()
        pltpu.make_async_copy(seg_hbm.at[pl.ds(d_j, C)], sb, ss).wait()
        pltpu.make_async_copy(att_hbm.at[pl.ds(d_j, C)], ab, sa).wait()

    def _process(j, hb, sb, ab):
        s_j, d_j = _chunk_base(j)
        lo = s_j - d_j
        hi = jnp.minimum(s_j + C, N) - d_j

        def _group(g, carry):
            sv = sb[pl.ds(g * L, L)]
            at16 = ab[pl.ds(g * L, L)]

            def _loads(k):
                r = g * L + k
                return [hb[r, pl.ds(L * q, L)] for q in range(D // L)]

            hk = _loads(0)
            for k in range(L):
                r = g * L + k
                s = sv[k]
                att = jnp.where((r >= lo) & (r < hi), at16[k], 0.0)
                av = jnp.full((L,), att, jnp.float32)
                pk = [av * hk[q] for q in range(D // L)]
                hk = _loads(k + 1) if k + 1 < L else None
                for q in range(D // L):
                    plsc.addupdate(num_acc.at[s, pl.ds(L * q, L)], pk[q])
                plsc.addupdate(den_acc.at[pl.ds(s * L, L)], av)
            return carry
        lax.fori_loop(0, C // L, _group, None)

    _dma_start(0, hb0, sb0, ab0, sh0, ss0, sa0)

    def _zero(r, carry):
        for k in range(D // L):
            num_acc[r, pl.ds(L * k, L)] = jnp.zeros((L,), jnp.float32)
        den_acc[pl.ds(r * L, L)] = jnp.zeros((L,), jnp.float32)
        return carry
    lax.fori_loop(0, G, _zero, None)

    def _outer(i, carry):
        j0 = 2 * i
        j1 = 2 * i + 1
        _dma_wait(j0, hb0, sb0, ab0, sh0, ss0, sa0)
        _dma_start(j1, hb1, sb1, ab1, sh1, ss1, sa1)
        _process(j0, hb0, sb0, ab0)
        _dma_wait(j1, hb1, sb1, ab1, sh1, ss1, sa1)
        _dma_start(jnp.minimum(j1 + 1, NCH - 1), hb0, sb0, ab0, sh0, ss0, sa0)
        _process(j1, hb1, sb1, ab1)
        return carry
    lax.fori_loop(0, NCH // 2, _outer, None)

    _dma_wait(NCH - 1, hb0, sb0, ab0, sh0, ss0, sa0)

    pltpu.sync_copy(num_acc, num_hbm.at[wid])
    pltpu.sync_copy(den_acc, den_hbm.at[wid])


GB = 64


def _tc_merge(p_ref, d_ref, o_ref):
    num = jnp.sum(p_ref[...], axis=0)
    dsum = jnp.sum(d_ref[...], axis=0)
    den_rep = jnp.reshape(
        jnp.broadcast_to(dsum[:, None, :], (GB // 8, 8, D)), (GB, D))
    rows = lax.broadcasted_iota(jnp.int32, (GB, D), 0)
    lanes = lax.broadcasted_iota(jnp.int32, (GB, D), 1)
    mask = lanes == (rows % 8) * L
    den = jnp.sum(jnp.where(mask, den_rep, 0.0), axis=1, keepdims=True)
    o_ref[...] = num / jnp.maximum(den, 1e-9)


def kernel(h, segment_ids, W, b):
    seg = segment_ids.astype(jnp.int32)
    wf = jnp.broadcast_to(W.astype(jnp.float32), (8, D))
    bf = b.astype(jnp.float32).reshape(1, 1)
    att = pl.pallas_call(
        _tc_att,
        grid=(NATT // BN,),
        in_specs=[
            pl.BlockSpec((BN, D), lambda i: (i, 0)),
            pl.BlockSpec((8, D), lambda i: (0, 0)),
            pl.BlockSpec((1, 1), lambda i: (0, 0)),
        ],
        out_specs=pl.BlockSpec((BN,), lambda i: (i,)),
        out_shape=jax.ShapeDtypeStruct((NATT,), jnp.float32),
    )(h, wf, bf)

    mesh = plsc.VectorSubcoreMesh(core_axis_name="c", subcore_axis_name="s")
    sc = pl.kernel(
        _sc_body,
        mesh=mesh,
        compiler_params=pltpu.CompilerParams(needs_layout_passes=False),
        out_type=(
            jax.ShapeDtypeStruct((NW, G, D), jnp.float32),
            jax.ShapeDtypeStruct((NW, G * L), jnp.float32),
        ),
        scratch_types=[
            pltpu.VMEM((C, D), jnp.float32),
            pltpu.VMEM((C, D), jnp.float32),
            pltpu.VMEM((C,), jnp.int32),
            pltpu.VMEM((C,), jnp.int32),
            pltpu.VMEM((C,), jnp.float32),
            pltpu.VMEM((C,), jnp.float32),
            pltpu.VMEM((G, D), jnp.float32),
            pltpu.VMEM((G * L,), jnp.float32),
            pltpu.SemaphoreType.DMA,
            pltpu.SemaphoreType.DMA,
            pltpu.SemaphoreType.DMA,
            pltpu.SemaphoreType.DMA,
            pltpu.SemaphoreType.DMA,
            pltpu.SemaphoreType.DMA,
        ],
    )
    num_p, den_p = sc(h, seg, att)
    den3 = den_p.reshape(NW, G // 8, D)
    out = pl.pallas_call(
        _tc_merge,
        grid=(G // GB,),
        in_specs=[
            pl.BlockSpec((NW, GB, D), lambda i: (0, i, 0)),
            pl.BlockSpec((NW, GB // 8, D), lambda i: (0, i, 0)),
        ],
        out_specs=pl.BlockSpec((GB, D), lambda i: (i, 0)),
        out_shape=jax.ShapeDtypeStruct((G, D), jnp.float32),
    )(num_p, den3)
    return out
